# Initial kernel scaffold; baseline (speedup 1.0000x reference)
import jax
import jax.numpy as jnp
from jax.experimental import pallas as pl


def kernel(x, grid0, grid1, grid2, W_label, b_label, W_rgb, b_rgb):
    raise NotImplementedError("write your pallas kernel here")


# speedup vs baseline: 995.8988x; 995.8988x over previous
import functools

import jax
import jax.numpy as jnp
import numpy as np
from jax import lax
from jax.experimental import pallas as pl
from jax.experimental.pallas import tpu as pltpu
from jax.experimental.pallas import tpu_sc as plsc

N_POINTS = 200
SIDE = 33
NV = SIDE ** 3
NV_PAD = 35952
NC, NS = 2, 16
NW = NC * NS
LANES = 16
B = 8192
RPT = B // NW
NG = RPT // LANES
TPAD = 208
CLIP_HI = np.float32(1.0 - 1e-6)
IDX_CHUNK = 128
N_CHUNKS = 8 * RPT // IDX_CHUNK

_CORNER_OFF = (0, 1, 33, 34, 1089, 1090, 1122, 1123)


def _spherical(theta, phi):
    st = jnp.sin(theta)
    return jnp.stack([st * jnp.cos(phi), st * jnp.sin(phi), jnp.cos(theta)], axis=-1)


def _interp_matrix(R):
    i = jnp.arange(SIDE, dtype=jnp.float32)
    pos = i * np.float32(R / 32.0)
    pi = jnp.clip(jnp.floor(pos).astype(jnp.int32), 0, R - 1)
    frac = pos - pi.astype(jnp.float32)
    lo = jax.nn.one_hot(pi, R + 1, dtype=jnp.float32) * (1.0 - frac)[:, None]
    hi = jax.nn.one_hot(pi + 1, R + 1, dtype=jnp.float32) * frac[:, None]
    return lo + hi


def _upsample(grid, R):
    g = grid.reshape(R + 1, R + 1, R + 1, grid.shape[1])
    W = _interp_matrix(R)
    g = jnp.einsum("ai,ijkf->ajkf", W, g)
    g = jnp.einsum("bj,ajkf->abkf", W, g)
    g = jnp.einsum("ck,abkf->abcf", W, g)
    return g.reshape(SIDE ** 3, grid.shape[1])


def _build_fused(grid0, grid1, grid2, W_label, b_label, W_rgb, b_rgb):
    U0 = _upsample(grid0, 8)
    U1 = _upsample(grid1, 16)
    Ucat = jnp.concatenate([U0, U1, grid2], axis=1)
    F0 = Ucat[:, ::4] @ W_label + b_label
    mask = np.ones(12, dtype=bool)
    mask[::4] = False
    Frgb = Ucat[:, mask] @ W_rgb + b_rgb
    return jnp.concatenate([F0, Frgb], axis=1)


def _sigmoid(x):
    return 1.0 / (1.0 + jnp.exp(-x))


def _sc_body(t_hbm, rdat_hbm, L_hbm, F_hbm, hits_hbm, rgbt_hbm,
             t_v, rdat_v, L_v, cidx_v, fsel_v, rows_v, hits_st, rgb_st, sem):
    wid = lax.axis_index("s") * NC + lax.axis_index("c")
    base = wid * RPT
    pltpu.sync_copy(t_hbm, t_v)
    pltpu.sync_copy(rdat_hbm.at[pl.ds(wid * 6 * RPT, 6 * RPT)], rdat_v)
    pltpu.sync_copy(L_hbm, L_v)
    iota = lax.iota(jnp.int32, LANES)

    for g in range(NG):
        sl = pl.ds(g * LANES, LANES)
        p1 = tuple(rdat_v[pl.ds(i * RPT + g * LANES, LANES)] for i in range(3))
        dd = tuple(rdat_v[pl.ds(i * RPT + g * LANES, LANES)] for i in range(3, 6))

        def cell(tj, p1=p1, dd=dd):
            pifs = []
            for p1c, dc in zip(p1, dd):
                u = jnp.minimum(jnp.maximum((p1c + dc * tj + 1.0) * 0.5, 0.0), CLIP_HI)
                pos = u * 32.0
                piv = pos.astype(jnp.int32)
                pifs.append((piv, pos - piv.astype(jnp.float32)))
            (pix, fx), (piy, fy), (piz, fz) = pifs
            return (pix * 33 + piy) * 33 + piz, fx, fy, fz

        def trilerp(vals, fx, fy, fz):
            a00 = vals[0] + (vals[1] - vals[0]) * fz
            a01 = vals[2] + (vals[3] - vals[2]) * fz
            a10 = vals[4] + (vals[5] - vals[4]) * fz
            a11 = vals[6] + (vals[7] - vals[6]) * fz
            b0 = a00 + (a01 - a00) * fy
            b1 = a10 + (a11 - a10) * fy
            return b0 + (b1 - b0) * fx

        def body(j, carry, cell=cell, trilerp=trilerp):
            vmax, vmin = carry
            tj = plsc.load_gather(t_v, [jnp.full((LANES,), j, jnp.int32)])
            idx0, fx, fy, fz = cell(tj)
            vals = [plsc.load_gather(L_v, [idx0 + off]) for off in _CORNER_OFF]
            lg = trilerp(vals, fx, fy, fz)
            vmax = jnp.maximum(vmax, lg)
            cand = jnp.where(lg > 0.0, jnp.full((LANES,), j, jnp.int32),
                             jnp.full((LANES,), N_POINTS, jnp.int32))
            return vmax, jnp.minimum(vmin, cand)

        init = (jnp.full((LANES,), -jnp.inf, jnp.float32),
                jnp.full((LANES,), N_POINTS, jnp.int32))
        vmax, vmin = lax.fori_loop(0, N_POINTS, body, init)

        hits_st[sl] = _sigmoid(vmax)
        idx_sel = jnp.where(vmin == N_POINTS, jnp.zeros((LANES,), jnp.int32), vmin)
        t_sel = plsc.load_gather(t_v, [idx_sel])
        idx0, fx, fy, fz = cell(t_sel)
        fsel_v[pl.ds(0 * RPT + g * LANES, LANES)] = fx
        fsel_v[pl.ds(1 * RPT + g * LANES, LANES)] = fy
        fsel_v[pl.ds(2 * RPT + g * LANES, LANES)] = fz
        for c, off in enumerate(_CORNER_OFF):
            cidx_v[pl.ds(c * RPT + g * LANES, LANES)] = idx0 + off

    descs = [pltpu.async_copy(F_hbm.at[cidx_v.at[pl.ds(ch * IDX_CHUNK, IDX_CHUNK)]],
                              rows_v.at[pl.ds(ch * IDX_CHUNK, IDX_CHUNK)], sem)
             for ch in range(N_CHUNKS)]
    for d in descs:
        d.wait()

    for g in range(NG):
        sl = pl.ds(g * LANES, LANES)
        fx = fsel_v[pl.ds(0 * RPT + g * LANES, LANES)]
        fy = fsel_v[pl.ds(1 * RPT + g * LANES, LANES)]
        fz = fsel_v[pl.ds(2 * RPT + g * LANES, LANES)]
        rbase = iota + g * LANES
        for k in range(3):
            kk = jnp.full((LANES,), k + 1, jnp.int32)
            vals = [plsc.load_gather(rows_v, [rbase + c * RPT, kk]) for c in range(8)]
            a00 = vals[0] + (vals[1] - vals[0]) * fz
            a01 = vals[2] + (vals[3] - vals[2]) * fz
            a10 = vals[4] + (vals[5] - vals[4]) * fz
            a11 = vals[6] + (vals[7] - vals[6]) * fz
            b0 = a00 + (a01 - a00) * fy
            b1 = a10 + (a11 - a10) * fy
            rgb_st[pl.ds(k * RPT + g * LANES, LANES)] = _sigmoid(b0 + (b1 - b0) * fx)

    pltpu.sync_copy(hits_st, hits_hbm.at[pl.ds(base, RPT)])
    for k in range(3):
        pltpu.sync_copy(rgb_st.at[pl.ds(k * RPT, RPT)],
                        rgbt_hbm.at[pl.ds(k * B + base, RPT)])


_sc_kernel = functools.partial(
    pl.kernel,
    out_type=(jax.ShapeDtypeStruct((B,), jnp.float32),
              jax.ShapeDtypeStruct((3 * B,), jnp.float32)),
    mesh=plsc.VectorSubcoreMesh(core_axis_name="c", subcore_axis_name="s",
                                num_cores=NC, num_subcores=NS),
    compiler_params=pltpu.CompilerParams(needs_layout_passes=False,
                                         use_tc_tiling_on_sc=False),
    scratch_types=[
        pltpu.VMEM((TPAD,), jnp.float32),
        pltpu.VMEM((6 * RPT,), jnp.float32),
        pltpu.VMEM((NV_PAD,), jnp.float32),
        pltpu.VMEM((8 * RPT,), jnp.int32),
        pltpu.VMEM((3 * RPT,), jnp.float32),
        pltpu.VMEM((8 * RPT, 4), jnp.float32),
        pltpu.VMEM((RPT,), jnp.float32),
        pltpu.VMEM((3 * RPT,), jnp.float32),
        pltpu.SemaphoreType.DMA,
    ],
)(_sc_body)


def kernel(x, grid0, grid1, grid2, W_label, b_label, W_rgb, b_rgb):
    t = jnp.linspace(0.0, 1.0, N_POINTS, dtype=jnp.float32)
    t_pad = jnp.concatenate([t, jnp.zeros((TPAD - N_POINTS,), jnp.float32)])
    p1 = _spherical(x[:, 0], x[:, 1])
    p2 = _spherical(x[:, 2], x[:, 3])
    d = p2 - p1
    rdat = jnp.concatenate([p1.T, d.T], axis=0)
    rdat_t = rdat.reshape(6, NW, RPT).transpose(1, 0, 2).reshape(NW * 6 * RPT)
    F = _build_fused(grid0, grid1, grid2, W_label, b_label, W_rgb, b_rgb)
    L = jnp.concatenate([F[:, 0], jnp.zeros((NV_PAD - NV,), jnp.float32)])
    hits_flat, rgb_t = _sc_kernel(t_pad, rdat_t, L, F)
    return hits_flat.reshape(B, 1), rgb_t.reshape(3, B).T


# speedup vs baseline: 1114.5919x; 1.1192x over previous
import functools

import jax
import jax.numpy as jnp
import numpy as np
from jax import lax
from jax.experimental import pallas as pl
from jax.experimental.pallas import tpu as pltpu
from jax.experimental.pallas import tpu_sc as plsc

N_POINTS = 200
SIDE = 33
NV = SIDE ** 3
NV_PAD = 35952
NC, NS = 2, 16
NW = NC * NS
LANES = 16
B = 8192
RPT = B // NW
NG = RPT // LANES
TPAD = 208
CLIP_HI = np.float32(1.0 - 1e-6)
IDX_CHUNK = 128
N_CHUNKS = 8 * RPT // IDX_CHUNK

_CORNER_OFF = (0, 1, 33, 34, 1089, 1090, 1122, 1123)


def _spherical(theta, phi):
    st = jnp.sin(theta)
    return jnp.stack([st * jnp.cos(phi), st * jnp.sin(phi), jnp.cos(theta)], axis=-1)


def _interp_matrix(R):
    i = jnp.arange(SIDE, dtype=jnp.float32)
    pos = i * np.float32(R / 32.0)
    pi = jnp.clip(jnp.floor(pos).astype(jnp.int32), 0, R - 1)
    frac = pos - pi.astype(jnp.float32)
    lo = jax.nn.one_hot(pi, R + 1, dtype=jnp.float32) * (1.0 - frac)[:, None]
    hi = jax.nn.one_hot(pi + 1, R + 1, dtype=jnp.float32) * frac[:, None]
    return lo + hi


def _upsample(grid, R):
    g = grid.reshape(R + 1, R + 1, R + 1, grid.shape[1])
    W = _interp_matrix(R)
    g = jnp.einsum("ai,ijkf->ajkf", W, g)
    g = jnp.einsum("bj,ajkf->abkf", W, g)
    g = jnp.einsum("ck,abkf->abcf", W, g)
    return g.reshape(SIDE ** 3, grid.shape[1])


def _build_fused(grid0, grid1, grid2, W_label, b_label, W_rgb, b_rgb):
    U0 = _upsample(grid0, 8)
    U1 = _upsample(grid1, 16)
    Ucat = jnp.concatenate([U0, U1, grid2], axis=1)
    F0 = Ucat[:, ::4] @ W_label + b_label
    mask = np.ones(12, dtype=bool)
    mask[::4] = False
    Frgb = Ucat[:, mask] @ W_rgb + b_rgb
    return jnp.concatenate(
        [F0, Frgb, jnp.zeros((NV, 12), jnp.float32)], axis=1)


def _sigmoid(x):
    return 1.0 / (1.0 + jnp.exp(-x))


def _sc_body(t_hbm, rdat_hbm, L_hbm, F_hbm, hits_hbm, rgbt_hbm,
             t_v, rdat_v, L_v, cidx_v, fsel_v, rows_v, hits_st, rgb_st, sem):
    wid = lax.axis_index("s") * NC + lax.axis_index("c")
    base = wid * RPT
    pltpu.sync_copy(t_hbm, t_v)
    pltpu.sync_copy(rdat_hbm.at[pl.ds(wid * 6 * RPT, 6 * RPT)], rdat_v)
    pltpu.sync_copy(L_hbm, L_v)
    iota = lax.iota(jnp.int32, LANES)

    for g in range(NG):
        sl = pl.ds(g * LANES, LANES)
        p1 = tuple(rdat_v[pl.ds(i * RPT + g * LANES, LANES)] for i in range(3))
        dd = tuple(rdat_v[pl.ds(i * RPT + g * LANES, LANES)] for i in range(3, 6))

        def cell(tj, p1=p1, dd=dd):
            pifs = []
            for p1c, dc in zip(p1, dd):
                u = jnp.minimum(jnp.maximum((p1c + dc * tj + 1.0) * 0.5, 0.0), CLIP_HI)
                pos = u * 32.0
                piv = pos.astype(jnp.int32)
                pifs.append((piv, pos - piv.astype(jnp.float32)))
            (pix, fx), (piy, fy), (piz, fz) = pifs
            return (pix * 33 + piy) * 33 + piz, fx, fy, fz

        def trilerp(vals, fx, fy, fz):
            a00 = vals[0] + (vals[1] - vals[0]) * fz
            a01 = vals[2] + (vals[3] - vals[2]) * fz
            a10 = vals[4] + (vals[5] - vals[4]) * fz
            a11 = vals[6] + (vals[7] - vals[6]) * fz
            b0 = a00 + (a01 - a00) * fy
            b1 = a10 + (a11 - a10) * fy
            return b0 + (b1 - b0) * fx

        def body(j, carry, cell=cell, trilerp=trilerp):
            vmax, vmin = carry
            tj = plsc.load_gather(t_v, [jnp.full((LANES,), j, jnp.int32)])
            idx0, fx, fy, fz = cell(tj)
            vals = [plsc.load_gather(L_v, [idx0 + off]) for off in _CORNER_OFF]
            lg = trilerp(vals, fx, fy, fz)
            vmax = jnp.maximum(vmax, lg)
            cand = jnp.where(lg > 0.0, jnp.full((LANES,), j, jnp.int32),
                             jnp.full((LANES,), N_POINTS, jnp.int32))
            return vmax, jnp.minimum(vmin, cand)

        init = (jnp.full((LANES,), -jnp.inf, jnp.float32),
                jnp.full((LANES,), N_POINTS, jnp.int32))
        vmax, vmin = lax.fori_loop(0, N_POINTS, body, init)

        hits_st[sl] = _sigmoid(vmax)
        idx_sel = jnp.where(vmin == N_POINTS, jnp.zeros((LANES,), jnp.int32), vmin)
        t_sel = plsc.load_gather(t_v, [idx_sel])
        idx0, fx, fy, fz = cell(t_sel)
        fsel_v[pl.ds(0 * RPT + g * LANES, LANES)] = fx
        fsel_v[pl.ds(1 * RPT + g * LANES, LANES)] = fy
        fsel_v[pl.ds(2 * RPT + g * LANES, LANES)] = fz
        for c, off in enumerate(_CORNER_OFF):
            cidx_v[pl.ds(c * RPT + g * LANES, LANES)] = idx0 + off

    descs = [pltpu.async_copy(F_hbm.at[cidx_v.at[pl.ds(ch * IDX_CHUNK, IDX_CHUNK)]],
                              rows_v.at[pl.ds(ch * IDX_CHUNK, IDX_CHUNK)], sem)
             for ch in range(N_CHUNKS)]
    for d in descs:
        d.wait()

    for g in range(NG):
        sl = pl.ds(g * LANES, LANES)
        fx = fsel_v[pl.ds(0 * RPT + g * LANES, LANES)]
        fy = fsel_v[pl.ds(1 * RPT + g * LANES, LANES)]
        fz = fsel_v[pl.ds(2 * RPT + g * LANES, LANES)]
        rbase = iota + g * LANES
        for k in range(3):
            kk = jnp.full((LANES,), k + 1, jnp.int32)
            vals = [plsc.load_gather(rows_v, [rbase + c * RPT, kk]) for c in range(8)]
            a00 = vals[0] + (vals[1] - vals[0]) * fz
            a01 = vals[2] + (vals[3] - vals[2]) * fz
            a10 = vals[4] + (vals[5] - vals[4]) * fz
            a11 = vals[6] + (vals[7] - vals[6]) * fz
            b0 = a00 + (a01 - a00) * fy
            b1 = a10 + (a11 - a10) * fy
            rgb_st[pl.ds(k * RPT + g * LANES, LANES)] = _sigmoid(b0 + (b1 - b0) * fx)

    pltpu.sync_copy(hits_st, hits_hbm.at[pl.ds(base, RPT)])
    for k in range(3):
        pltpu.sync_copy(rgb_st.at[pl.ds(k * RPT, RPT)],
                        rgbt_hbm.at[pl.ds(k * B + base, RPT)])


_sc_kernel = functools.partial(
    pl.kernel,
    out_type=(jax.ShapeDtypeStruct((B,), jnp.float32),
              jax.ShapeDtypeStruct((3 * B,), jnp.float32)),
    mesh=plsc.VectorSubcoreMesh(core_axis_name="c", subcore_axis_name="s",
                                num_cores=NC, num_subcores=NS),
    compiler_params=pltpu.CompilerParams(needs_layout_passes=False,
                                         use_tc_tiling_on_sc=False),
    scratch_types=[
        pltpu.VMEM((TPAD,), jnp.float32),
        pltpu.VMEM((6 * RPT,), jnp.float32),
        pltpu.VMEM((NV_PAD,), jnp.float32),
        pltpu.VMEM((8 * RPT,), jnp.int32),
        pltpu.VMEM((3 * RPT,), jnp.float32),
        pltpu.VMEM((8 * RPT, 16), jnp.float32),
        pltpu.VMEM((RPT,), jnp.float32),
        pltpu.VMEM((3 * RPT,), jnp.float32),
        pltpu.SemaphoreType.DMA,
    ],
)(_sc_body)


def kernel(x, grid0, grid1, grid2, W_label, b_label, W_rgb, b_rgb):
    t = jnp.linspace(0.0, 1.0, N_POINTS, dtype=jnp.float32)
    t_pad = jnp.concatenate([t, jnp.zeros((TPAD - N_POINTS,), jnp.float32)])
    p1 = _spherical(x[:, 0], x[:, 1])
    p2 = _spherical(x[:, 2], x[:, 3])
    d = p2 - p1
    rdat = jnp.concatenate([p1.T, d.T], axis=0)
    rdat_t = rdat.reshape(6, NW, RPT).transpose(1, 0, 2).reshape(NW * 6 * RPT)
    F = _build_fused(grid0, grid1, grid2, W_label, b_label, W_rgb, b_rgb)
    L = jnp.concatenate([F[:, 0], jnp.zeros((NV_PAD - NV,), jnp.float32)])
    hits_flat, rgb_t = _sc_kernel(t_pad, rdat_t, L, F)
    return hits_flat.reshape(B, 1), rgb_t.reshape(3, B).T
